# baseline (speedup 1.0000x reference)
import math

import jax
import jax.numpy as jnp
from jax import lax
from jax.experimental import pallas as pl
from jax.experimental.pallas import tpu as pltpu
from jax.experimental.pallas import tpu_sc as plsc

EMBED_DIM = 256
MAX_LEN = 2048
SEQ_LEN = 2048
BATCH = 64
NUM_PEAKS = 50
PEAK_PAD = 64
SBLK = 128
NBLK = SEQ_LEN // SBLK
HEAD_BLKS = 3

NUM_CORES = 2
NUM_SUBCORES = 16
NUM_TILES = NUM_CORES * NUM_SUBCORES
ROWS_PER_TILE = SEQ_LEN // NUM_TILES
WORDS_PER_TILE = ROWS_PER_TILE * BATCH
NTGT = BATCH * NUM_PEAKS
NCHUNK = NTGT // 16


def _pe_table(max_len, dim):
    position = jnp.arange(0, max_len, dtype=jnp.float32)[:, None]
    div_term = jnp.exp(
        jnp.arange(0, dim, 2, dtype=jnp.float32) * (-math.log(1000.0) / dim))
    pe = jnp.zeros((max_len, dim), dtype=jnp.float32)
    pe = pe.at[:, 0::2].set(jnp.sin(position * div_term))
    pe = pe.at[:, 1::2].set(jnp.cos(position * div_term))
    return pe


def _sc_mask_body(tgt_hbm, zeros_hbm, mask_hbm, tgt_v, mask_v, sem1, sem2):
    wid = lax.axis_index("s") * NUM_CORES + lax.axis_index("c")
    lo = wid * WORDS_PER_TILE
    cp1 = pltpu.async_copy(tgt_hbm, tgt_v, sem1)
    cp2 = pltpu.async_copy(zeros_hbm, mask_v, sem2)
    cp1.wait()
    cp2.wait()

    ones16 = jnp.ones((16,), jnp.float32)

    @plsc.parallel_loop(0, NCHUNK, unroll=8)
    def _(c):
        local = tgt_v[pl.ds(c * 16, 16)] - lo
        valid = (local >= 0) & (local < WORDS_PER_TILE)
        plsc.store_scatter(mask_v, [local], ones16, mask=valid)

    pltpu.sync_copy(mask_v, mask_hbm.at[pl.ds(lo, WORDS_PER_TILE)])


def _sc_mask(targets, zeros):
    mesh = plsc.VectorSubcoreMesh(
        core_axis_name="c", subcore_axis_name="s", num_cores=NUM_CORES)
    run = pl.kernel(
        _sc_mask_body,
        mesh=mesh,
        out_type=jax.ShapeDtypeStruct((SEQ_LEN * BATCH,), jnp.float32),
        scratch_types=[
            pltpu.VMEM((NTGT,), jnp.int32),
            pltpu.VMEM((WORDS_PER_TILE,), jnp.float32),
            pltpu.SemaphoreType.DMA,
            pltpu.SemaphoreType.DMA,
        ],
        compiler_params=pltpu.CompilerParams(needs_layout_passes=False),
    )
    return run(targets, zeros).reshape(SEQ_LEN, BATCH)


def _tc_head_body(peaks_ref, x_ref, pe_ref, tab_ref, out_ref):
    i = pl.program_id(0)
    s_ids = jax.lax.broadcasted_iota(jnp.int32, (SBLK, 1, 1), 0) + i * SBLK
    peaks = peaks_ref[...]
    mask = jnp.any(peaks[None, :, :] == s_ids, axis=2)
    out_ref[...] = (
        x_ref[...]
        + pe_ref[...][:, None, :]
        + mask[:, :, None].astype(jnp.float32) * tab_ref[...][:, None, :]
    )


def _tc_head(x, pe, table, peaks):
    seq, batch, dim = x.shape
    return pl.pallas_call(
        _tc_head_body,
        grid=(HEAD_BLKS,),
        in_specs=[
            pl.BlockSpec((BATCH, PEAK_PAD), lambda i: (0, 0)),
            pl.BlockSpec((SBLK, BATCH, EMBED_DIM), lambda i: (i, 0, 0)),
            pl.BlockSpec((SBLK, EMBED_DIM), lambda i: (i, 0)),
            pl.BlockSpec((SBLK, EMBED_DIM), lambda i: (i, 0)),
        ],
        out_specs=pl.BlockSpec((SBLK, BATCH, EMBED_DIM), lambda i: (i, 0, 0)),
        out_shape=jax.ShapeDtypeStruct((seq, batch, dim), jnp.float32),
    )(peaks, x, pe, table)


def _tc_tail_body(partial_ref, x_ref, pe_ref, tab_ref, mask_ref, out_ref):
    del partial_ref
    out_ref[...] = (
        x_ref[...]
        + pe_ref[...][:, None, :]
        + mask_ref[...][:, :, None] * tab_ref[...][:, None, :]
    )


def _tc_tail(partial, x, pe, table, mask):
    seq, batch, dim = x.shape
    h = HEAD_BLKS
    return pl.pallas_call(
        _tc_tail_body,
        grid=(NBLK - HEAD_BLKS,),
        in_specs=[
            pl.BlockSpec(memory_space=pl.ANY),
            pl.BlockSpec((SBLK, BATCH, EMBED_DIM), lambda i: (i + h, 0, 0)),
            pl.BlockSpec((SBLK, EMBED_DIM), lambda i: (i + h, 0)),
            pl.BlockSpec((SBLK, EMBED_DIM), lambda i: (i + h, 0)),
            pl.BlockSpec((SBLK, BATCH), lambda i: (i + h, 0)),
        ],
        out_specs=pl.BlockSpec((SBLK, BATCH, EMBED_DIM), lambda i: (i + h, 0, 0)),
        out_shape=jax.ShapeDtypeStruct((seq, batch, dim), jnp.float32),
        input_output_aliases={0: 0},
    )(partial, x, pe, table, mask)


@jax.jit
def _run(x, targets, zeros, peaks, table, pe):
    mask = _sc_mask(targets, zeros)
    head = _tc_head(x, pe, table, peaks)
    return _tc_tail(head, x, pe, table, mask)


def kernel(x, peak_positions, table):
    seq, batch, dim = x.shape
    pe = _pe_table(seq, dim)
    pp = peak_positions.astype(jnp.int32)
    valid = (pp >= 0) & (pp < seq)
    raw = pp * BATCH + jnp.arange(batch, dtype=jnp.int32)[:, None]
    targets = jnp.where(valid, raw, jnp.int32(2**30)).reshape(-1)
    zeros = jnp.zeros((WORDS_PER_TILE,), jnp.float32)
    peaks = jnp.pad(
        pp, ((0, 0), (0, PEAK_PAD - pp.shape[1])), constant_values=-1)
    return _run(x, targets, zeros, peaks, table, pe)


# speedup vs baseline: 1.1129x; 1.1129x over previous
import math

import jax
import jax.numpy as jnp
import numpy as np
from jax import lax
from jax.experimental import pallas as pl
from jax.experimental.pallas import tpu as pltpu
from jax.experimental.pallas import tpu_sc as plsc

EMBED_DIM = 256
MAX_LEN = 2048
SEQ_LEN = 2048
BATCH = 64
NUM_PEAKS = 50
PEAK_PAD = 64
MASK_LANES = 128
SBLK = 128
NBLK = SEQ_LEN // SBLK
HEAD_BLKS = 2

NUM_CORES = 2
NUM_SUBCORES = 16
NUM_TILES = NUM_CORES * NUM_SUBCORES
ROWS_PER_TILE = SEQ_LEN // NUM_TILES
WORDS_PER_TILE = ROWS_PER_TILE * MASK_LANES
NTGT = BATCH * NUM_PEAKS
NCHUNK = NTGT // 16


def _pe_const(max_len, dim):
    position = np.arange(0, max_len, dtype=np.float32)[:, None]
    div_term = np.exp(
        np.arange(0, dim, 2, dtype=np.float32) * (-math.log(1000.0) / dim))
    pe = np.zeros((max_len, dim), dtype=np.float32)
    pe[:, 0::2] = np.sin(position * div_term)
    pe[:, 1::2] = np.cos(position * div_term)
    return jnp.asarray(pe)


def _sc_mask_body(tgt_hbm, zeros_hbm, mask_hbm, tgt_v, mask_v, sem1, sem2):
    wid = lax.axis_index("s") * NUM_CORES + lax.axis_index("c")
    lo = wid * WORDS_PER_TILE
    cp1 = pltpu.async_copy(tgt_hbm, tgt_v, sem1)
    cp2 = pltpu.async_copy(zeros_hbm, mask_v, sem2)
    cp1.wait()
    cp2.wait()

    ones16 = jnp.ones((16,), jnp.float32)

    @plsc.parallel_loop(0, NCHUNK, unroll=8)
    def _(c):
        local = tgt_v[pl.ds(c * 16, 16)] - lo
        valid = (local >= 0) & (local < WORDS_PER_TILE)
        plsc.store_scatter(mask_v, [local], ones16, mask=valid)

    pltpu.sync_copy(mask_v, mask_hbm.at[pl.ds(lo, WORDS_PER_TILE)])


def _sc_mask(targets, zeros):
    mesh = plsc.VectorSubcoreMesh(
        core_axis_name="c", subcore_axis_name="s", num_cores=NUM_CORES)
    run = pl.kernel(
        _sc_mask_body,
        mesh=mesh,
        out_type=jax.ShapeDtypeStruct((SEQ_LEN * MASK_LANES,), jnp.float32),
        scratch_types=[
            pltpu.VMEM((NTGT,), jnp.int32),
            pltpu.VMEM((WORDS_PER_TILE,), jnp.float32),
            pltpu.SemaphoreType.DMA,
            pltpu.SemaphoreType.DMA,
        ],
        compiler_params=pltpu.CompilerParams(needs_layout_passes=False),
    )
    return run(targets, zeros).reshape(SEQ_LEN, MASK_LANES)


def _tc_head_body(peaks_ref, x_ref, pe_ref, tab_ref, out_ref):
    i = pl.program_id(0)
    s_ids = jax.lax.broadcasted_iota(jnp.int32, (SBLK, 1, 1), 0) + i * SBLK
    peaks = peaks_ref[...]
    mask = jnp.any(peaks[None, :, :] == s_ids, axis=2)
    out_ref[...] = (
        x_ref[...]
        + pe_ref[...][:, None, :]
        + mask[:, :, None].astype(jnp.float32) * tab_ref[...][:, None, :]
    )


def _tc_head(x, pe, table, peaks):
    seq, batch, dim = x.shape
    return pl.pallas_call(
        _tc_head_body,
        grid=(HEAD_BLKS,),
        in_specs=[
            pl.BlockSpec((BATCH, PEAK_PAD), lambda i: (0, 0)),
            pl.BlockSpec((SBLK, BATCH, EMBED_DIM), lambda i: (i, 0, 0)),
            pl.BlockSpec((SBLK, EMBED_DIM), lambda i: (i, 0)),
            pl.BlockSpec((SBLK, EMBED_DIM), lambda i: (i, 0)),
        ],
        out_specs=pl.BlockSpec((SBLK, BATCH, EMBED_DIM), lambda i: (i, 0, 0)),
        out_shape=jax.ShapeDtypeStruct((seq, batch, dim), jnp.float32),
    )(peaks, x, pe, table)


def _tc_tail_body(partial_ref, x_ref, pe_ref, tab_ref, mask_ref, out_ref):
    del partial_ref
    mask = mask_ref[...][:, :BATCH]
    out_ref[...] = (
        x_ref[...]
        + pe_ref[...][:, None, :]
        + mask[:, :, None] * tab_ref[...][:, None, :]
    )


def _tc_tail(partial, x, pe, table, mask):
    seq, batch, dim = x.shape
    h = HEAD_BLKS
    return pl.pallas_call(
        _tc_tail_body,
        grid=(NBLK - HEAD_BLKS,),
        in_specs=[
            pl.BlockSpec(memory_space=pl.ANY),
            pl.BlockSpec((SBLK, BATCH, EMBED_DIM), lambda i: (i + h, 0, 0)),
            pl.BlockSpec((SBLK, EMBED_DIM), lambda i: (i + h, 0)),
            pl.BlockSpec((SBLK, EMBED_DIM), lambda i: (i + h, 0)),
            pl.BlockSpec((SBLK, MASK_LANES), lambda i: (i + h, 0)),
        ],
        out_specs=pl.BlockSpec((SBLK, BATCH, EMBED_DIM), lambda i: (i + h, 0, 0)),
        out_shape=jax.ShapeDtypeStruct((seq, batch, dim), jnp.float32),
        input_output_aliases={0: 0},
    )(partial, x, pe, table, mask)


@jax.jit
def _run(x, targets, zeros, peaks, table, pe):
    mask = _sc_mask(targets, zeros)
    head = _tc_head(x, pe, table, peaks)
    return _tc_tail(head, x, pe, table, mask)


def kernel(x, peak_positions, table):
    seq, batch, dim = x.shape
    pe = _pe_const(seq, dim)
    pp = peak_positions.astype(jnp.int32)
    valid = (pp >= 0) & (pp < seq)
    raw = pp * MASK_LANES + jnp.arange(batch, dtype=jnp.int32)[:, None]
    targets = jnp.where(valid, raw, jnp.int32(2**30)).reshape(-1)
    zeros = jnp.zeros((WORDS_PER_TILE,), jnp.float32)
    peaks = jnp.pad(
        pp, ((0, 0), (0, PEAK_PAD - pp.shape[1])), constant_values=-1)
    return _run(x, targets, zeros, peaks, table, pe)


# speedup vs baseline: 1.1206x; 1.0069x over previous
import math

import jax
import jax.numpy as jnp
import numpy as np
from jax import lax
from jax.experimental import pallas as pl
from jax.experimental.pallas import tpu as pltpu
from jax.experimental.pallas import tpu_sc as plsc

EMBED_DIM = 256
MAX_LEN = 2048
SEQ_LEN = 2048
BATCH = 64
NUM_PEAKS = 50
PEAK_PAD = 64
MASK_LANES = 128
SBLK = 128
NBLK = SEQ_LEN // SBLK
HEAD_BLKS = 2

NUM_CORES = 2
NUM_SUBCORES = 16
NUM_TILES = NUM_CORES * NUM_SUBCORES
ROWS_PER_TILE = SEQ_LEN // NUM_TILES
WORDS_PER_TILE = ROWS_PER_TILE * MASK_LANES
NTGT = BATCH * NUM_PEAKS
NCHUNK = NTGT // 16


def _pe_const(max_len, dim):
    position = np.arange(0, max_len, dtype=np.float32)[:, None]
    div_term = np.exp(
        np.arange(0, dim, 2, dtype=np.float32) * (-math.log(1000.0) / dim))
    pe = np.zeros((max_len, dim), dtype=np.float32)
    pe[:, 0::2] = np.sin(position * div_term)
    pe[:, 1::2] = np.cos(position * div_term)
    return jnp.asarray(pe)


def _sc_mask_body(tgt_hbm, zeros_hbm, mask_hbm, tgt_v, mask_v, sem1, sem2):
    wid = lax.axis_index("s") * NUM_CORES + lax.axis_index("c")
    row_lo = wid * ROWS_PER_TILE
    lo = row_lo * MASK_LANES
    cp1 = pltpu.async_copy(tgt_hbm, tgt_v, sem1)
    cp2 = pltpu.async_copy(zeros_hbm, mask_v, sem2)
    cp1.wait()
    cp2.wait()

    ones16 = jnp.ones((16,), jnp.float32)

    def scan_body(c, carry):
        local = tgt_v[pl.ds(c * 16, 16)] - lo
        valid = (local >= 0) & (local < WORDS_PER_TILE)
        row = lax.shift_right_logical(local, 7)
        col = lax.bitwise_and(local, MASK_LANES - 1)
        plsc.store_scatter(mask_v, [row, col], ones16, mask=valid)
        return carry

    lax.fori_loop(0, NCHUNK, scan_body, 0)
    pltpu.sync_copy(mask_v, mask_hbm.at[pl.ds(row_lo, ROWS_PER_TILE), :])


def _sc_mask(targets, zeros):
    mesh = plsc.VectorSubcoreMesh(
        core_axis_name="c", subcore_axis_name="s", num_cores=NUM_CORES)
    run = pl.kernel(
        _sc_mask_body,
        mesh=mesh,
        out_type=jax.ShapeDtypeStruct((SEQ_LEN, MASK_LANES), jnp.float32),
        scratch_types=[
            pltpu.VMEM((NTGT,), jnp.int32),
            pltpu.VMEM((ROWS_PER_TILE, MASK_LANES), jnp.float32),
            pltpu.SemaphoreType.DMA,
            pltpu.SemaphoreType.DMA,
        ],
        compiler_params=pltpu.CompilerParams(needs_layout_passes=False),
    )
    return run(targets, zeros)


def _tc_head_body(peaks_ref, x_ref, pe_ref, tab_ref, out_ref):
    i = pl.program_id(0)
    s_ids = jax.lax.broadcasted_iota(jnp.int32, (SBLK, 1, 1), 0) + i * SBLK
    peaks = peaks_ref[...]
    mask = jnp.any(peaks[None, :, :] == s_ids, axis=2)
    out_ref[...] = (
        x_ref[...]
        + pe_ref[...][:, None, :]
        + mask[:, :, None].astype(jnp.float32) * tab_ref[...][:, None, :]
    )


def _tc_head(x, pe, table, peaks):
    seq, batch, dim = x.shape
    return pl.pallas_call(
        _tc_head_body,
        grid=(HEAD_BLKS,),
        in_specs=[
            pl.BlockSpec((BATCH, PEAK_PAD), lambda i: (0, 0)),
            pl.BlockSpec((SBLK, BATCH, EMBED_DIM), lambda i: (i, 0, 0)),
            pl.BlockSpec((SBLK, EMBED_DIM), lambda i: (i, 0)),
            pl.BlockSpec((SBLK, EMBED_DIM), lambda i: (i, 0)),
        ],
        out_specs=pl.BlockSpec((SBLK, BATCH, EMBED_DIM), lambda i: (i, 0, 0)),
        out_shape=jax.ShapeDtypeStruct((seq, batch, dim), jnp.float32),
    )(peaks, x, pe, table)


def _tc_tail_body(partial_ref, x_ref, pe_ref, tab_ref, mask_ref, out_ref):
    del partial_ref
    mask = mask_ref[...][:, :BATCH]
    out_ref[...] = (
        x_ref[...]
        + pe_ref[...][:, None, :]
        + mask[:, :, None] * tab_ref[...][:, None, :]
    )


def _tc_tail(partial, x, pe, table, mask):
    seq, batch, dim = x.shape
    h = HEAD_BLKS
    return pl.pallas_call(
        _tc_tail_body,
        grid=(NBLK - HEAD_BLKS,),
        in_specs=[
            pl.BlockSpec(memory_space=pl.ANY),
            pl.BlockSpec((SBLK, BATCH, EMBED_DIM), lambda i: (i + h, 0, 0)),
            pl.BlockSpec((SBLK, EMBED_DIM), lambda i: (i + h, 0)),
            pl.BlockSpec((SBLK, EMBED_DIM), lambda i: (i + h, 0)),
            pl.BlockSpec((SBLK, MASK_LANES), lambda i: (i + h, 0)),
        ],
        out_specs=pl.BlockSpec((SBLK, BATCH, EMBED_DIM), lambda i: (i + h, 0, 0)),
        out_shape=jax.ShapeDtypeStruct((seq, batch, dim), jnp.float32),
        input_output_aliases={0: 0},
    )(partial, x, pe, table, mask)


@jax.jit
def _run(x, targets, zeros, peaks, table, pe):
    mask = _sc_mask(targets, zeros)
    head = _tc_head(x, pe, table, peaks)
    return _tc_tail(head, x, pe, table, mask)


def kernel(x, peak_positions, table):
    seq, batch, dim = x.shape
    pe = _pe_const(seq, dim)
    pp = peak_positions.astype(jnp.int32)
    valid = (pp >= 0) & (pp < seq)
    raw = pp * MASK_LANES + jnp.arange(batch, dtype=jnp.int32)[:, None]
    targets = jnp.where(valid, raw, jnp.int32(2**30)).reshape(-1)
    zeros = jnp.asarray(np.zeros((ROWS_PER_TILE, MASK_LANES), np.float32))
    peaks = jnp.pad(
        pp, ((0, 0), (0, PEAK_PAD - pp.shape[1])), constant_values=-1)
    return _run(x, targets, zeros, peaks, table, pe)


# speedup vs baseline: 1.3577x; 1.2116x over previous
import functools
import math

import jax
import jax.numpy as jnp
import numpy as np
from jax.experimental import pallas as pl

EMBED_DIM = 256
MAX_LEN = 2048
SEQ_LEN = 2048
BATCH = 64
PEAK_PAD = 64
SBLK = 128


def _pe_const(max_len, dim):
    position = np.arange(0, max_len, dtype=np.float32)[:, None]
    div_term = np.exp(
        np.arange(0, dim, 2, dtype=np.float32) * (-math.log(1000.0) / dim))
    pe = np.zeros((max_len, dim), dtype=np.float32)
    pe[:, 0::2] = np.sin(position * div_term)
    pe[:, 1::2] = np.cos(position * div_term)
    return jnp.asarray(pe)


def _tc_body(peaks_ref, x_ref, pe_ref, tab_ref, out_ref):
    i = pl.program_id(0)
    s_ids = jax.lax.broadcasted_iota(jnp.int32, (SBLK, 1, 1), 0) + i * SBLK
    peaks = peaks_ref[...]
    mask = jnp.any(peaks[None, :, :] == s_ids, axis=2)
    out_ref[...] = (
        x_ref[...]
        + pe_ref[...][:, None, :]
        + mask[:, :, None].astype(jnp.float32) * tab_ref[...][:, None, :]
    )


@jax.jit
def _run(x, peaks, table, pe):
    seq, batch, dim = x.shape
    return pl.pallas_call(
        _tc_body,
        grid=(seq // SBLK,),
        in_specs=[
            pl.BlockSpec((BATCH, PEAK_PAD), lambda i: (0, 0)),
            pl.BlockSpec((SBLK, BATCH, EMBED_DIM), lambda i: (i, 0, 0)),
            pl.BlockSpec((SBLK, EMBED_DIM), lambda i: (i, 0)),
            pl.BlockSpec((SBLK, EMBED_DIM), lambda i: (i, 0)),
        ],
        out_specs=pl.BlockSpec((SBLK, BATCH, EMBED_DIM), lambda i: (i, 0, 0)),
        out_shape=jax.ShapeDtypeStruct((seq, batch, dim), jnp.float32),
    )(peaks, x, pe, table)


def kernel(x, peak_positions, table):
    seq, batch, dim = x.shape
    pe = _pe_const(seq, dim)
    peaks = jnp.pad(
        peak_positions.astype(jnp.int32),
        ((0, 0), (0, PEAK_PAD - peak_positions.shape[1])),
        constant_values=-1,
    )
    return _run(x, peaks, table, pe)
